# Initial kernel scaffold; baseline (speedup 1.0000x reference)
import jax
import jax.numpy as jnp
from jax.experimental import pallas as pl


def kernel(x, Wr, Wg, Wu, Wd):
    raise NotImplementedError("write your pallas kernel here")


# speedup vs baseline: 1.5701x; 1.5701x over previous
import functools

import jax
import jax.numpy as jnp
from jax import lax
from jax.experimental import pallas as pl
from jax.experimental.pallas import tpu as pltpu
from jax.experimental.pallas import tpu_sc as plsc

T = 2048
D = 768
F = 2048
NE = 8
TOPK = 2
BLK = 128
NB = T * TOPK // BLK + NE
NR = NB * BLK
AUXC = 0.01

NC = 2
NS = 16
NW = NC * NS
A_PER_W = T * TOPK // NW
T_PER_W = T // NW


def _excl_cumsum_ax0(s):
    c = s
    sh = 1
    n = s.shape[0]
    while sh < n:
        z = jnp.zeros((sh, s.shape[1]), s.dtype)
        c = c + jnp.concatenate([z, c[: n - sh, :]], axis=0)
        sh *= 2
    return c - s


def _excl_cumsum_ax1(v):
    c = v
    sh = 1
    n = v.shape[1]
    while sh < n:
        z = jnp.zeros((v.shape[0], sh), v.dtype)
        c = c + jnp.concatenate([z, c[:, : n - sh]], axis=1)
        sh *= 2
    return c - v


def _router_body(x_ref, wr_ref, dst_ref, tw_ref, blk_ref, aux_ref):
    xv = x_ref[...]
    wr = wr_ref[...]
    logits = lax.dot_general(xv, wr, (((1,), (1,)), ((), ())),
                             preferred_element_type=jnp.float32)
    m = jnp.max(logits, axis=-1, keepdims=True)
    el = jnp.exp(logits - m)
    probs = el / jnp.sum(el, axis=-1, keepdims=True)
    iota = lax.broadcasted_iota(jnp.int32, (T, NE), 1)
    m1 = jnp.max(probs, axis=-1, keepdims=True)
    i1 = jnp.min(jnp.where(probs == m1, iota, NE), axis=-1, keepdims=True)
    oh1 = iota == i1
    probs2 = jnp.where(oh1, -jnp.inf, probs)
    m2 = jnp.max(probs2, axis=-1, keepdims=True)
    i2 = jnp.min(jnp.where(probs2 == m2, iota, NE), axis=-1, keepdims=True)
    oh2 = iota == i2
    ssum = m1 + m2
    tw_ref[...] = jnp.concatenate([m1 / ssum, m2 / ssum], axis=1)

    oh1f = oh1.astype(jnp.float32)
    oh2f = oh2.astype(jnp.float32)
    assign = oh1f + oh2f
    prod = (jnp.sum(assign, axis=0, keepdims=True)
            * jnp.sum(probs, axis=0, keepdims=True))
    aux_ref[...] = (NE * AUXC / (T * T)) * jnp.sum(prod, axis=1, keepdims=True)

    c = _excl_cumsum_ax0(assign)
    rank1 = jnp.sum(c * oh1f, axis=-1, keepdims=True)
    rank2 = jnp.sum(c * oh2f, axis=-1, keepdims=True)
    counts = jnp.sum(assign, axis=0, keepdims=True)
    padded = jnp.floor((counts + (BLK - 1)) * (1.0 / BLK)) * BLK
    offs = _excl_cumsum_ax1(padded)
    off1 = jnp.sum(offs * oh1f, axis=-1, keepdims=True)
    off2 = jnp.sum(offs * oh2f, axis=-1, keepdims=True)
    dst_ref[...] = jnp.concatenate(
        [off1 + rank1, off2 + rank2], axis=1).astype(jnp.int32)

    ends = jnp.reshape(offs + padded, (NE, 1))
    bio = lax.broadcasted_iota(jnp.int32, (NE, 128), 1).astype(
        jnp.float32) * float(BLK)
    blk = jnp.sum((ends <= bio).astype(jnp.float32), axis=0, keepdims=True)
    blk_ref[...] = jnp.minimum(blk, NE - 1).astype(jnp.int32)


def _ffn_body(be_ref, xs_ref, wg_ref, wu_ref, wd_ref, yo_ref):
    xb = xs_ref[...]
    hg = jnp.dot(xb, wg_ref[0], preferred_element_type=jnp.float32)
    hu = jnp.dot(xb, wu_ref[0], preferred_element_type=jnp.float32)
    h = hg * jax.nn.sigmoid(hg) * hu
    yo_ref[...] = jnp.dot(h, wd_ref[0], preferred_element_type=jnp.float32)


def _combine_body(a_ref, b_ref, tw_ref, y_ref):
    y_ref[...] = (tw_ref[:, 0:1] * a_ref[...] + tw_ref[:, 1:2] * b_ref[...])


@functools.cache
def _sc_kernels():
    vmesh = plsc.VectorSubcoreMesh(core_axis_name="c", subcore_axis_name="s")

    @functools.partial(
        pl.kernel, mesh=vmesh,
        out_type=jax.ShapeDtypeStruct((NR, D), jnp.float32),
        scratch_types=[
            pltpu.VMEM((A_PER_W,), jnp.int32),
            pltpu.VMEM((A_PER_W,), jnp.int32),
            pltpu.VMEM((A_PER_W, D), jnp.float32),
            pltpu.SemaphoreType.DMA,
        ])
    def sc_dispatch(x_hbm, dup_hbm, dst_hbm, xs_hbm, sidx_v, didx_v, rows_v, sem):
        wid = lax.axis_index("s") * NC + lax.axis_index("c")
        base = wid * A_PER_W
        pltpu.sync_copy(dup_hbm.at[pl.ds(base, A_PER_W)], sidx_v)
        pltpu.sync_copy(dst_hbm.at[pl.ds(base, A_PER_W)], didx_v)
        pltpu.async_copy(x_hbm.at[sidx_v], rows_v, sem).wait()
        pltpu.sync_copy(rows_v, xs_hbm.at[didx_v])

    @functools.partial(
        pl.kernel, mesh=vmesh,
        out_type=[jax.ShapeDtypeStruct((T, D), jnp.float32),
                  jax.ShapeDtypeStruct((T, D), jnp.float32)],
        scratch_types=[
            pltpu.VMEM((T_PER_W,), jnp.int32),
            pltpu.VMEM((T_PER_W, D), jnp.float32),
            pltpu.SemaphoreType.DMA,
        ])
    def sc_gather(yo_hbm, d0_hbm, d1_hbm, a_hbm, b_hbm, idx_v, rows_v, sem):
        wid = lax.axis_index("s") * NC + lax.axis_index("c")
        base = wid * T_PER_W
        pltpu.sync_copy(d0_hbm.at[pl.ds(base, T_PER_W)], idx_v)
        pltpu.async_copy(yo_hbm.at[idx_v], rows_v, sem).wait()
        pltpu.sync_copy(rows_v, a_hbm.at[pl.ds(base, T_PER_W)])
        pltpu.sync_copy(d1_hbm.at[pl.ds(base, T_PER_W)], idx_v)
        pltpu.async_copy(yo_hbm.at[idx_v], rows_v, sem).wait()
        pltpu.sync_copy(rows_v, b_hbm.at[pl.ds(base, T_PER_W)])

    return sc_dispatch, sc_gather


def kernel(x, Wr, Wg, Wu, Wd):
    b, s, d = x.shape
    xt = x.reshape(b * s, d)

    dst, topw, blk, aux = pl.pallas_call(
        _router_body,
        out_shape=[
            jax.ShapeDtypeStruct((T, TOPK), jnp.int32),
            jax.ShapeDtypeStruct((T, TOPK), jnp.float32),
            jax.ShapeDtypeStruct((1, 128), jnp.int32),
            jax.ShapeDtypeStruct((1, 1), jnp.float32),
        ],
    )(xt, Wr)

    sc_dispatch, sc_gather = _sc_kernels()
    dup = jnp.repeat(jnp.arange(T, dtype=jnp.int32), TOPK)
    xs = sc_dispatch(xt, dup, dst.reshape(-1))

    grid_spec = pltpu.PrefetchScalarGridSpec(
        num_scalar_prefetch=1,
        grid=(NB,),
        in_specs=[
            pl.BlockSpec((BLK, D), lambda i, be: (i, 0)),
            pl.BlockSpec((1, D, F), lambda i, be: (be[i], 0, 0)),
            pl.BlockSpec((1, D, F), lambda i, be: (be[i], 0, 0)),
            pl.BlockSpec((1, F, D), lambda i, be: (be[i], 0, 0)),
        ],
        out_specs=pl.BlockSpec((BLK, D), lambda i, be: (i, 0)),
    )
    yo = pl.pallas_call(
        _ffn_body,
        grid_spec=grid_spec,
        out_shape=jax.ShapeDtypeStruct((NR, D), jnp.float32),
    )(blk.reshape(-1), xs, Wg, Wu, Wd)

    a_g, b_g = sc_gather(yo, dst[:, 0], dst[:, 1])

    y = pl.pallas_call(
        _combine_body,
        grid=(T // 256,),
        in_specs=[
            pl.BlockSpec((256, D), lambda i: (i, 0)),
            pl.BlockSpec((256, D), lambda i: (i, 0)),
            pl.BlockSpec((256, TOPK), lambda i: (i, 0)),
        ],
        out_specs=pl.BlockSpec((256, D), lambda i: (i, 0)),
        out_shape=jax.ShapeDtypeStruct((T, D), jnp.float32),
    )(a_g, b_g, topw)

    return y.reshape(b, s, d), aux.reshape(())
